# baseline (speedup 1.0000x reference)
import functools
import math

import jax
import jax.numpy as jnp
from jax import lax
from jax.experimental import pallas as pl
from jax.experimental.pallas import tpu as pltpu
from jax.experimental.pallas import tpu_sc as plsc

D_MODEL = 64
SCALE = math.sqrt(D_MODEL)

_NC = 2
_NS = 16
_NW = _NC * _NS

_IDXW = 128
_CHUNK = 512
_JROWS = _CHUNK // _IDXW


@functools.lru_cache(maxsize=None)
def _build(batch: int, vocab: int, d_model: int):
    assert d_model == D_MODEL
    assert batch % (_NW * _CHUNK) == 0
    rows_per_w = batch // _NW
    n_chunks = rows_per_w // _CHUNK
    idx_rows_per_w = rows_per_w // _IDXW

    def body(x_hbm, table_hbm, out_hbm, idx_v, rows_v, sem):
        wid = lax.axis_index("s") * _NC + lax.axis_index("c")

        def chunk_body(i, carry):
            irow0 = wid * idx_rows_per_w + i * _JROWS
            r0 = wid * rows_per_w + i * _CHUNK
            pltpu.sync_copy(x_hbm.at[pl.ds(irow0, _JROWS)], idx_v)
            copies = [
                pltpu.async_copy(
                    table_hbm.at[idx_v.at[j]],
                    rows_v.at[pl.ds(j * _IDXW, _IDXW)],
                    sem,
                )
                for j in range(_JROWS)
            ]
            for c in copies:
                c.wait()

            def scale_body(r, c2):
                for k in range(d_model // 16):
                    rows_v[r, pl.ds(k * 16, 16)] = (
                        rows_v[r, pl.ds(k * 16, 16)] * SCALE
                    )
                return c2

            lax.fori_loop(0, _CHUNK, scale_body, 0)
            pltpu.sync_copy(rows_v, out_hbm.at[pl.ds(r0, _CHUNK)])
            return carry

        lax.fori_loop(0, n_chunks, chunk_body, 0)

    return pl.kernel(
        body,
        out_type=jax.ShapeDtypeStruct((batch, d_model), jnp.float32),
        scratch_types=[
            pltpu.VMEM((_JROWS, _IDXW), jnp.int32),
            pltpu.VMEM((_CHUNK, D_MODEL), jnp.float32),
            pltpu.SemaphoreType.DMA,
        ],
        mesh=plsc.VectorSubcoreMesh(core_axis_name="c", subcore_axis_name="s"),
        compiler_params=pltpu.CompilerParams(use_tc_tiling_on_sc=False),
    )


def kernel(x, table):
    s0, s1 = x.shape
    batch = s0 * s1
    vocab, d_model = table.shape
    xf = x.reshape(batch // _IDXW, _IDXW).astype(jnp.int32)
    out = _build(batch, vocab, d_model)(xf, table)
    return out.reshape(s0, s1, d_model)


# speedup vs baseline: 1.1374x; 1.1374x over previous
import functools
import math

import jax
import jax.numpy as jnp
from jax import lax
from jax.experimental import pallas as pl
from jax.experimental.pallas import tpu as pltpu
from jax.experimental.pallas import tpu_sc as plsc

D_MODEL = 64
SCALE = math.sqrt(D_MODEL)

_NC = 2
_NS = 16
_NW = _NC * _NS

_IDXW = 128
_CHUNK = 256
_JROWS = _CHUNK // _IDXW
_NBUF = 4
_SROWS = 4


@functools.lru_cache(maxsize=None)
def _build(batch: int, vocab: int, d_model: int):
    assert d_model == D_MODEL
    assert batch % (_NW * _CHUNK * _NBUF) == 0
    rows_per_w = batch // _NW
    n_chunks = rows_per_w // _CHUNK
    idx_rows_per_w = rows_per_w // _IDXW

    def body(x_hbm, table_hbm, out_hbm, idx_v, bufs, gsems, wsems):
        wid = lax.axis_index("s") * _NC + lax.axis_index("c")
        out_base = wid * rows_per_w

        pltpu.sync_copy(x_hbm.at[pl.ds(wid * idx_rows_per_w, idx_rows_per_w)],
                        idx_v)

        def fire_gather(c, slot):
            for j in range(_JROWS):
                pltpu.async_copy(
                    table_hbm.at[idx_v.at[c * _JROWS + j]],
                    bufs[slot].at[pl.ds(j * _IDXW, _IDXW)],
                    gsems[slot],
                )

        def drain_gather(c, slot):
            for j in range(_JROWS):
                pltpu.make_async_copy(
                    table_hbm.at[idx_v.at[c * _JROWS + j]],
                    bufs[slot].at[pl.ds(j * _IDXW, _IDXW)],
                    gsems[slot],
                ).wait()

        def out_ref(c, slot):
            return (bufs[slot], out_hbm.at[pl.ds(out_base + c * _CHUNK,
                                                 _CHUNK)])

        for b in range(_NBUF - 1):
            fire_gather(b, b)

        def outer(o, carry):
            for b in range(_NBUF):
                c = o * _NBUF + b
                drain_gather(c, b)

                def scale_body(r, c2):
                    for u in range(_SROWS):
                        row = r * _SROWS + u
                        for k in range(d_model // 16):
                            bufs[b][row, pl.ds(k * 16, 16)] = (
                                bufs[b][row, pl.ds(k * 16, 16)] * SCALE
                            )
                    return c2

                lax.fori_loop(0, _CHUNK // _SROWS, scale_body, 0)

                src, dst = out_ref(c, b)
                pltpu.async_copy(src, dst, wsems[b])

                pb = (b + _NBUF - 1) % _NBUF

                @pl.when(c > 0)
                def _():
                    psrc, pdst = out_ref(c - 1, pb)
                    pltpu.make_async_copy(psrc, pdst, wsems[pb]).wait()

                @pl.when(c + _NBUF - 1 < n_chunks)
                def _():
                    fire_gather(c + _NBUF - 1, pb)
            return carry

        lax.fori_loop(0, n_chunks // _NBUF, outer, 0)

        lsrc, ldst = out_ref(n_chunks - 1, (n_chunks - 1) % _NBUF)
        pltpu.make_async_copy(lsrc, ldst, wsems[(n_chunks - 1) % _NBUF]).wait()

    return pl.kernel(
        body,
        out_type=jax.ShapeDtypeStruct((batch, d_model), jnp.float32),
        scratch_types=[
            pltpu.VMEM((idx_rows_per_w, _IDXW), jnp.int32),
            [pltpu.VMEM((_CHUNK, D_MODEL), jnp.float32)
             for _ in range(_NBUF)],
            [pltpu.SemaphoreType.DMA for _ in range(_NBUF)],
            [pltpu.SemaphoreType.DMA for _ in range(_NBUF)],
        ],
        mesh=plsc.VectorSubcoreMesh(core_axis_name="c", subcore_axis_name="s"),
        compiler_params=pltpu.CompilerParams(use_tc_tiling_on_sc=False),
    )


def kernel(x, table):
    s0, s1 = x.shape
    batch = s0 * s1
    vocab, d_model = table.shape
    xf = x.reshape(batch // _IDXW, _IDXW).astype(jnp.int32)
    out = _build(batch, vocab, d_model)(xf, table)
    return out.reshape(s0, s1, d_model)
